# baseline (speedup 1.0000x reference)
import jax
import jax.numpy as jnp
from jax.experimental import pallas as pl

_NUM_CLASSES = 1000
_B4 = 32


def _onehot_body(x_ref, o_ref):
    iota = jax.lax.broadcasted_iota(jnp.int32, (26, _NUM_CLASSES), 1)
    for b in range(_B4):
        idx_col = x_ref[0, :, b:b + 1]
        o_ref[b] = (iota == idx_col).astype(jnp.float32)


def kernel(x):
    x = x.astype(jnp.int32)
    batch, feats = x.shape
    nb = batch // _B4
    x3 = x.reshape(nb, _B4, feats).transpose(0, 2, 1)
    out = pl.pallas_call(
        _onehot_body,
        grid=(nb,),
        in_specs=[pl.BlockSpec((1, feats, _B4), lambda j: (j, 0, 0))],
        out_specs=pl.BlockSpec((_B4, feats, _NUM_CLASSES), lambda j: (j, 0, 0)),
        out_shape=jax.ShapeDtypeStruct((batch, feats, _NUM_CLASSES), jnp.float32),
    )(x3)
    return out


# speedup vs baseline: 1.0009x; 1.0009x over previous
import jax
import jax.numpy as jnp
from jax import lax
from jax.experimental import pallas as pl
from jax.experimental.pallas import tpu as pltpu

_NUM_CLASSES = 1000
_B4 = 32
_NBUF = 4


def _onehot_body(x_ref, o_hbm, scratch, sems):
    j = pl.program_id(0)
    nb = pl.num_programs(0)
    slot = lax.rem(j, _NBUF)

    @pl.when(j >= _NBUF)
    def _wait_slot():
        pltpu.make_async_copy(
            scratch.at[slot],
            o_hbm.at[pl.ds((j - _NBUF) * _B4, _B4)],
            sems.at[slot],
        ).wait()

    iota = jax.lax.broadcasted_iota(jnp.int32, (26, _NUM_CLASSES), 1)
    for b in range(_B4):
        idx_col = x_ref[0, :, b:b + 1]
        scratch[slot, b] = (iota == idx_col).astype(jnp.float32)

    pltpu.make_async_copy(
        scratch.at[slot],
        o_hbm.at[pl.ds(j * _B4, _B4)],
        sems.at[slot],
    ).start()

    @pl.when(j == nb - 1)
    def _drain():
        for s in range(_NBUF):
            pltpu.make_async_copy(
                scratch.at[s],
                o_hbm.at[pl.ds(0, _B4)],
                sems.at[s],
            ).wait()


def kernel(x):
    x = x.astype(jnp.int32)
    batch, feats = x.shape
    nb = batch // _B4
    x3 = x.reshape(nb, _B4, feats).transpose(0, 2, 1)
    out = pl.pallas_call(
        _onehot_body,
        grid=(nb,),
        in_specs=[pl.BlockSpec((1, feats, _B4), lambda j: (j, 0, 0))],
        out_specs=pl.BlockSpec(memory_space=pltpu.MemorySpace.HBM),
        out_shape=jax.ShapeDtypeStruct((batch, feats, _NUM_CLASSES), jnp.float32),
        scratch_shapes=[
            pltpu.VMEM((_NBUF, _B4, feats, _NUM_CLASSES), jnp.float32),
            pltpu.SemaphoreType.DMA((_NBUF,)),
        ],
    )(x3)
    return out


# speedup vs baseline: 4.6529x; 4.6489x over previous
import jax
import jax.numpy as jnp
from jax.experimental import pallas as pl

_NUM_CLASSES = 1000


def _onehot_body(x_ref, o_ref):
    row = x_ref[0]
    cls = jax.lax.broadcasted_iota(jnp.int32, (_NUM_CLASSES, 4096), 0)
    o_ref[0] = (cls == row).astype(jnp.float32)


def kernel(x):
    x = x.astype(jnp.int32)
    batch, feats = x.shape
    x_t = x.T.reshape(feats, 1, batch)
    out_phys = pl.pallas_call(
        _onehot_body,
        grid=(feats,),
        in_specs=[pl.BlockSpec((1, 1, batch), lambda f: (f, 0, 0))],
        out_specs=pl.BlockSpec((1, _NUM_CLASSES, batch), lambda f: (f, 0, 0)),
        out_shape=jax.ShapeDtypeStruct((feats, _NUM_CLASSES, batch), jnp.float32),
    )(x_t)
    return out_phys.transpose(2, 0, 1)


# speedup vs baseline: 4.8017x; 1.0320x over previous
import jax
import jax.numpy as jnp
from jax.experimental import pallas as pl

_NUM_CLASSES = 1000
_CBLK = 200


def _onehot_body(x_ref, o_ref):
    f = pl.program_id(0)
    k = pl.program_id(1)
    row = x_ref[pl.ds(f, 1), :]
    cls = jax.lax.broadcasted_iota(jnp.int32, (_CBLK, 4096), 0) + k * _CBLK
    o_ref[0] = (cls == row).astype(jnp.float32)


def kernel(x):
    x = x.astype(jnp.int32)
    batch, feats = x.shape
    x_t = x.T
    out_phys = pl.pallas_call(
        _onehot_body,
        grid=(feats, _NUM_CLASSES // _CBLK),
        in_specs=[pl.BlockSpec((feats, batch), lambda f, k: (0, 0))],
        out_specs=pl.BlockSpec((1, _CBLK, batch), lambda f, k: (f, k, 0)),
        out_shape=jax.ShapeDtypeStruct((feats, _NUM_CLASSES, batch), jnp.float32),
    )(x_t)
    return out_phys.transpose(2, 0, 1)
